# baseline (speedup 1.0000x reference)
import jax
import jax.numpy as jnp
from jax import lax
from jax.experimental import pallas as pl
from jax.experimental.pallas import tpu as pltpu
from jax.experimental.pallas import tpu_sc as plsc

N_NODES = 10000
N_EDGES = 320000
FEATS = 128

NC = 2
NS = 16
LANES = 16

EDGES_PER_TILE = N_EDGES // (NC * NS)
CHUNK = 50
STEPS = EDGES_PER_TILE // CHUNK
NSLOT = 5
STAGE = 25
NBLK = STEPS // STAGE
NCYC = STAGE // NSLOT
ACC_ROWS = 10240
ROWS_PER_SUB = ACC_ROWS // NS
ZCH = 40


def _sc_aggregate_kernel(x_hbm, e_hbm, part_hbm, *refs):
    rows = refs[0:NSLOT]
    src_v = refs[NSLOT]
    dst_v = refs[NSLOT + 1]
    acc_sh = refs[NSLOT + 2]
    rsem = refs[NSLOT + 3:2 * NSLOT + 3]

    c = lax.axis_index("c")
    s = lax.axis_index("s")

    @pl.loop(0, ZCH)
    def _zero_rows(r):
        @pl.loop(0, FEATS, step=LANES)
        def _zero_lanes(col):
            rows[0][r, pl.ds(col, LANES)] = jnp.zeros((LANES,), jnp.float32)

    for n in range(ROWS_PER_SUB // ZCH):
        pltpu.async_copy(rows[0].at[pl.ds(0, ZCH)],
                         acc_sh.at[pl.ds(s * ROWS_PER_SUB + n * ZCH, ZCH)],
                         rsem[n % NSLOT])
    for n in range(ROWS_PER_SUB // ZCH):
        pltpu.make_async_copy(rows[0].at[pl.ds(0, ZCH)],
                              acc_sh.at[pl.ds(0, ZCH)],
                              rsem[n % NSLOT]).wait()

    plsc.subcore_barrier()

    def sg(k, j):
        pltpu.async_copy(x_hbm.at[src_v.at[j]], rows[k], rsem[k])

    def wg(k):
        pltpu.make_async_copy(x_hbm.at[src_v.at[0]], rows[k], rsem[k]).wait()

    def ss(k, j):
        pltpu.async_copy(rows[k], acc_sh.at[dst_v.at[j]], rsem[k], add=True)

    def ws(k):
        pltpu.make_async_copy(rows[k], acc_sh.at[dst_v.at[0]], rsem[k]).wait()

    @pl.loop(0, NBLK)
    def _block(b):
        pltpu.sync_copy(e_hbm.at[0, c, s, b], src_v)
        pltpu.sync_copy(e_hbm.at[1, c, s, b], dst_v)

        for j in range(3):
            sg(j, j)
        wg(0); ss(0, 0); sg(3, 3)
        for j in range(1, NSLOT):
            wg(j); ss(j, j); ws(j - 1); sg((j + 3) % NSLOT, j + 3)

        @pl.loop(1, NCYC - 1)
        def _cycle(i):
            j0 = i * NSLOT
            for u in range(NSLOT):
                wg(u); ss(u, j0 + u); ws((u + 4) % NSLOT)
                sg((u + 3) % NSLOT, j0 + u + 3)

        j0 = (NCYC - 1) * NSLOT
        for u in range(NSLOT):
            wg(u); ss(u, j0 + u); ws((u + 4) % NSLOT)
            if u + 3 < NSLOT:
                sg((u + 3) % NSLOT, j0 + u + 3)
        ws(NSLOT - 1)

    plsc.subcore_barrier()

    @pl.when(s < NS - 1)
    def _full():
        pltpu.sync_copy(acc_sh.at[pl.ds(s * ROWS_PER_SUB, ROWS_PER_SUB)],
                        part_hbm.at[c, pl.ds(s * ROWS_PER_SUB, ROWS_PER_SUB)])

    @pl.when(s == NS - 1)
    def _tail():
        last = N_NODES - (NS - 1) * ROWS_PER_SUB
        pltpu.sync_copy(acc_sh.at[pl.ds((NS - 1) * ROWS_PER_SUB, last)],
                        part_hbm.at[c, pl.ds((NS - 1) * ROWS_PER_SUB, last)])


def _sc_aggregate(x, eidx):
    mesh = plsc.VectorSubcoreMesh(core_axis_name="c", subcore_axis_name="s")
    scratch = (
        [pltpu.VMEM((CHUNK, FEATS), jnp.float32)] * NSLOT
        + [pltpu.VMEM((STAGE, CHUNK), jnp.int32)] * 2
        + [pltpu.VMEM_SHARED((ACC_ROWS, FEATS), jnp.float32)]
        + [pltpu.SemaphoreType.DMA] * NSLOT
    )
    kern = pl.kernel(
        _sc_aggregate_kernel,
        out_type=jax.ShapeDtypeStruct((NC, ACC_ROWS, FEATS), jnp.float32),
        mesh=mesh,
        scratch_types=scratch,
    )
    return kern(x, eidx)


def _mm_kernel(p0_ref, p1_ref, w_ref, o_ref):
    x = p0_ref[...] + p1_ref[...]
    o_ref[...] = jnp.dot(x, w_ref[...], preferred_element_type=jnp.float32)


ROW_BLOCK = 2000


def _combine_matmul(partials, weight):
    return pl.pallas_call(
        _mm_kernel,
        grid=(N_NODES // ROW_BLOCK,),
        in_specs=[
            pl.BlockSpec((ROW_BLOCK, FEATS), lambda i: (i, 0)),
            pl.BlockSpec((ROW_BLOCK, FEATS), lambda i: (i, 0)),
            pl.BlockSpec((FEATS, FEATS), lambda i: (0, 0)),
        ],
        out_specs=pl.BlockSpec((ROW_BLOCK, FEATS), lambda i: (i, 0)),
        out_shape=jax.ShapeDtypeStruct((N_NODES, FEATS), jnp.float32),
    )(partials[0], partials[1], weight)


@jax.jit
def kernel(inputs, edge_index, weight):
    e6 = edge_index.astype(jnp.int32).reshape(2, NC, NS, NBLK, STAGE, CHUNK)
    partials = _sc_aggregate(inputs, e6)
    return _combine_matmul(partials, weight)


# speedup vs baseline: 1.0477x; 1.0477x over previous
import jax
import jax.numpy as jnp
from jax import lax
from jax.experimental import pallas as pl
from jax.experimental.pallas import tpu as pltpu
from jax.experimental.pallas import tpu_sc as plsc

N_NODES = 10000
N_EDGES = 320000
FEATS = 128

NC = 2
NS = 16
LANES = 16

EDGES_PER_TILE = N_EDGES // (NC * NS)
CHUNK = 50
STEPS = EDGES_PER_TILE // CHUNK
NSLOT = 5
STAGE = 25
NBLK = STEPS // STAGE
NCYC = STAGE // NSLOT
ACC_ROWS = 10240
ROWS_PER_SUB = ACC_ROWS // NS
ZCH = 40


def _sc_aggregate_kernel(x_hbm, e_hbm, part_hbm, *refs):
    rows = refs[0:NSLOT]
    src_v = refs[NSLOT]
    dst_v = refs[NSLOT + 1]
    acc_sh = refs[NSLOT + 2]
    rsem = refs[NSLOT + 3:2 * NSLOT + 3]

    c = lax.axis_index("c")
    s = lax.axis_index("s")

    @pl.loop(0, ZCH)
    def _zero_rows(r):
        @pl.loop(0, FEATS, step=LANES)
        def _zero_lanes(col):
            rows[0][r, pl.ds(col, LANES)] = jnp.zeros((LANES,), jnp.float32)

    for n in range(ROWS_PER_SUB // ZCH):
        pltpu.async_copy(rows[0].at[pl.ds(0, ZCH)],
                         acc_sh.at[pl.ds(s * ROWS_PER_SUB + n * ZCH, ZCH)],
                         rsem[n % NSLOT])
    for n in range(ROWS_PER_SUB // ZCH):
        pltpu.make_async_copy(rows[0].at[pl.ds(0, ZCH)],
                              acc_sh.at[pl.ds(0, ZCH)],
                              rsem[n % NSLOT]).wait()

    plsc.subcore_barrier()

    def sg(k, j):
        pltpu.async_copy(x_hbm.at[src_v.at[j]], rows[k], rsem[k])

    def wg(k):
        pltpu.make_async_copy(x_hbm.at[src_v.at[0]], rows[k], rsem[k]).wait()

    def ss(k, j):
        pltpu.async_copy(rows[k], acc_sh.at[dst_v.at[j]], rsem[k], add=True)

    def ws(k):
        pltpu.make_async_copy(rows[k], acc_sh.at[dst_v.at[0]], rsem[k]).wait()

    @pl.loop(0, NBLK)
    def _block(b):
        pltpu.sync_copy(e_hbm.at[0, c, s, b], src_v)
        pltpu.sync_copy(e_hbm.at[1, c, s, b], dst_v)

        for j in range(3):
            sg(j, j)
        wg(0); ss(0, 0); sg(3, 3)
        for j in range(1, NSLOT):
            wg(j); ss(j, j); ws(j - 1); sg((j + 3) % NSLOT, j + 3)

        @pl.loop(1, NCYC - 1)
        def _cycle(i):
            j0 = i * NSLOT
            for u in range(NSLOT):
                wg(u); ss(u, j0 + u); ws((u + 4) % NSLOT)
                sg((u + 3) % NSLOT, j0 + u + 3)

        j0 = (NCYC - 1) * NSLOT
        for u in range(NSLOT):
            wg(u); ss(u, j0 + u); ws((u + 4) % NSLOT)
            if u + 3 < NSLOT:
                sg((u + 3) % NSLOT, j0 + u + 3)
        ws(NSLOT - 1)

    plsc.subcore_barrier()

    @pl.when(s < NS - 1)
    def _full():
        pltpu.sync_copy(acc_sh.at[pl.ds(s * ROWS_PER_SUB, ROWS_PER_SUB)],
                        part_hbm.at[c, pl.ds(s * ROWS_PER_SUB, ROWS_PER_SUB)])

    @pl.when(s == NS - 1)
    def _tail():
        last = N_NODES - (NS - 1) * ROWS_PER_SUB
        pltpu.sync_copy(acc_sh.at[pl.ds((NS - 1) * ROWS_PER_SUB, last)],
                        part_hbm.at[c, pl.ds((NS - 1) * ROWS_PER_SUB, last)])


def _sc_aggregate(x, eidx):
    mesh = plsc.VectorSubcoreMesh(core_axis_name="c", subcore_axis_name="s")
    scratch = (
        [pltpu.VMEM((CHUNK, FEATS), jnp.float32)] * NSLOT
        + [pltpu.VMEM((STAGE, CHUNK), jnp.int32)] * 2
        + [pltpu.VMEM_SHARED((ACC_ROWS, FEATS), jnp.float32)]
        + [pltpu.SemaphoreType.DMA] * NSLOT
    )
    kern = pl.kernel(
        _sc_aggregate_kernel,
        out_type=jax.ShapeDtypeStruct((NC, ACC_ROWS, FEATS), jnp.float32),
        mesh=mesh,
        scratch_types=scratch,
    )
    return kern(x, eidx)


def _mm_kernel(p_ref, w_ref, o_ref):
    x = p_ref[0] + p_ref[1]
    o_ref[...] = jnp.dot(x, w_ref[...], preferred_element_type=jnp.float32)


ROW_BLOCK = 2000


def _combine_matmul(partials, weight):
    return pl.pallas_call(
        _mm_kernel,
        grid=(N_NODES // ROW_BLOCK,),
        in_specs=[
            pl.BlockSpec((NC, ROW_BLOCK, FEATS), lambda i: (0, i, 0)),
            pl.BlockSpec((FEATS, FEATS), lambda i: (0, 0)),
        ],
        out_specs=pl.BlockSpec((ROW_BLOCK, FEATS), lambda i: (i, 0)),
        out_shape=jax.ShapeDtypeStruct((N_NODES, FEATS), jnp.float32),
    )(partials, weight)


@jax.jit
def kernel(inputs, edge_index, weight):
    e6 = edge_index.astype(jnp.int32).reshape(2, NC, NS, NBLK, STAGE, CHUNK)
    partials = _sc_aggregate(inputs, e6)
    return _combine_matmul(partials, weight)


# speedup vs baseline: 1.0568x; 1.0087x over previous
import jax
import jax.numpy as jnp
from jax import lax
from jax.experimental import pallas as pl
from jax.experimental.pallas import tpu as pltpu
from jax.experimental.pallas import tpu_sc as plsc

N_NODES = 10000
N_EDGES = 320000
FEATS = 128

NC = 2
NS = 16
LANES = 16

EDGES_PER_TILE = N_EDGES // (NC * NS)
CHUNK = 50
STEPS = EDGES_PER_TILE // CHUNK
NSLOT = 5
STAGE = 25
NBLK = STEPS // STAGE
NCYC = STAGE // NSLOT
ACC_ROWS = 10240
ROWS_PER_SUB = ACC_ROWS // NS
ZCH = 40


def _sc_aggregate_kernel(x_hbm, e_hbm, part_hbm, *refs):
    rows = refs[0:NSLOT]
    src_v = refs[NSLOT]
    dst_v = refs[NSLOT + 1]
    acc_sh = refs[NSLOT + 2]
    rsem = refs[NSLOT + 3:2 * NSLOT + 3]

    c = lax.axis_index("c")
    s = lax.axis_index("s")

    def sg(k, j):
        pltpu.async_copy(x_hbm.at[src_v.at[j]], rows[k], rsem[k])

    def wg(k):
        pltpu.make_async_copy(x_hbm.at[src_v.at[0]], rows[k], rsem[k]).wait()

    def ss(k, j):
        pltpu.async_copy(rows[k], acc_sh.at[dst_v.at[j]], rsem[k], add=True)

    def ws(k):
        pltpu.make_async_copy(rows[k], acc_sh.at[dst_v.at[0]], rsem[k]).wait()

    pltpu.sync_copy(e_hbm.at[0, c, s, 0], src_v)
    pltpu.sync_copy(e_hbm.at[1, c, s, 0], dst_v)
    for j in range(3):
        sg(j, j)

    @pl.loop(0, ZCH)
    def _zero_rows(r):
        @pl.loop(0, FEATS, step=LANES)
        def _zero_lanes(col):
            rows[NSLOT - 1][r, pl.ds(col, LANES)] = (
                jnp.zeros((LANES,), jnp.float32))

    for n in range(ROWS_PER_SUB // ZCH):
        pltpu.async_copy(rows[NSLOT - 1].at[pl.ds(0, ZCH)],
                         acc_sh.at[pl.ds(s * ROWS_PER_SUB + n * ZCH, ZCH)],
                         rsem[NSLOT - 1])
    for n in range(ROWS_PER_SUB // ZCH):
        pltpu.make_async_copy(rows[NSLOT - 1].at[pl.ds(0, ZCH)],
                              acc_sh.at[pl.ds(0, ZCH)],
                              rsem[NSLOT - 1]).wait()

    plsc.subcore_barrier()

    @pl.loop(0, NBLK)
    def _block(b):
        @pl.when(b > 0)
        def _stage():
            pltpu.sync_copy(e_hbm.at[0, c, s, b], src_v)
            pltpu.sync_copy(e_hbm.at[1, c, s, b], dst_v)
            for j in range(3):
                sg(j, j)

        wg(0); ss(0, 0); sg(3, 3)
        for j in range(1, NSLOT):
            wg(j); ss(j, j); ws(j - 1); sg((j + 3) % NSLOT, j + 3)

        @pl.loop(1, NCYC - 1)
        def _cycle(i):
            j0 = i * NSLOT
            for u in range(NSLOT):
                wg(u); ss(u, j0 + u)
                sg((u + 3) % NSLOT, j0 + u + 3)
                ws((u + 4) % NSLOT)

        j0 = (NCYC - 1) * NSLOT
        for u in range(NSLOT):
            wg(u); ss(u, j0 + u); ws((u + 4) % NSLOT)
            if u + 3 < NSLOT:
                sg((u + 3) % NSLOT, j0 + u + 3)
        ws(NSLOT - 1)

    plsc.subcore_barrier()

    @pl.when(s < NS - 1)
    def _full():
        pltpu.sync_copy(acc_sh.at[pl.ds(s * ROWS_PER_SUB, ROWS_PER_SUB)],
                        part_hbm.at[c, pl.ds(s * ROWS_PER_SUB, ROWS_PER_SUB)])

    @pl.when(s == NS - 1)
    def _tail():
        last = N_NODES - (NS - 1) * ROWS_PER_SUB
        pltpu.sync_copy(acc_sh.at[pl.ds((NS - 1) * ROWS_PER_SUB, last)],
                        part_hbm.at[c, pl.ds((NS - 1) * ROWS_PER_SUB, last)])


def _sc_aggregate(x, eidx):
    mesh = plsc.VectorSubcoreMesh(core_axis_name="c", subcore_axis_name="s")
    scratch = (
        [pltpu.VMEM((CHUNK, FEATS), jnp.float32)] * NSLOT
        + [pltpu.VMEM((STAGE, CHUNK), jnp.int32)] * 2
        + [pltpu.VMEM_SHARED((ACC_ROWS, FEATS), jnp.float32)]
        + [pltpu.SemaphoreType.DMA] * NSLOT
    )
    kern = pl.kernel(
        _sc_aggregate_kernel,
        out_type=jax.ShapeDtypeStruct((NC, ACC_ROWS, FEATS), jnp.float32),
        mesh=mesh,
        scratch_types=scratch,
    )
    return kern(x, eidx)


def _mm_kernel(p_ref, w_ref, o_ref):
    x = p_ref[0] + p_ref[1]
    o_ref[...] = jnp.dot(x, w_ref[...], preferred_element_type=jnp.float32)


ROW_BLOCK = 2000


def _combine_matmul(partials, weight):
    return pl.pallas_call(
        _mm_kernel,
        grid=(N_NODES // ROW_BLOCK,),
        in_specs=[
            pl.BlockSpec((NC, ROW_BLOCK, FEATS), lambda i: (0, i, 0)),
            pl.BlockSpec((FEATS, FEATS), lambda i: (0, 0)),
        ],
        out_specs=pl.BlockSpec((ROW_BLOCK, FEATS), lambda i: (i, 0)),
        out_shape=jax.ShapeDtypeStruct((N_NODES, FEATS), jnp.float32),
    )(partials, weight)


@jax.jit
def kernel(inputs, edge_index, weight):
    e6 = edge_index.astype(jnp.int32).reshape(2, NC, NS, NBLK, STAGE, CHUNK)
    partials = _sc_aggregate(inputs, e6)
    return _combine_matmul(partials, weight)
